# baseline (speedup 1.0000x reference)
import jax
import jax.numpy as jnp
from jax.experimental import pallas as pl


def _router_kernel(x_ref, w_ref, b_ref, probs_ref, logits_ref, z_ref):
    xb = x_ref[...].astype(jnp.bfloat16)
    wb = w_ref[...].astype(jnp.bfloat16)
    logits = jax.lax.dot_general(
        xb, wb,
        dimension_numbers=(((1,), (0,)), ((), ())),
        preferred_element_type=jnp.float32,
    )
    logits = logits + b_ref[...]
    logits_ref[...] = logits
    m = jnp.max(logits, axis=-1, keepdims=True)
    e = jnp.exp(logits - m)
    s = jnp.sum(e, axis=-1, keepdims=True)
    probs_ref[...] = e / s
    log_z = jnp.log(s) + m
    z_ref[...] = jnp.sum(log_z * log_z).reshape(1, 1, 1)


def kernel(token_inputs, W, b, num_experts, expert_capacity):
    G, T, H = token_inputs.shape
    E = W.shape[1]
    M = G * T
    x = token_inputs.reshape(M, H)
    BM = 1024
    N = M // BM

    probs, logits, zparts = pl.pallas_call(
        _router_kernel,
        grid=(N,),
        in_specs=[
            pl.BlockSpec((BM, H), lambda i: (i, 0)),
            pl.BlockSpec((H, E), lambda i: (0, 0)),
            pl.BlockSpec((1, E), lambda i: (0, 0)),
        ],
        out_specs=[
            pl.BlockSpec((BM, E), lambda i: (i, 0)),
            pl.BlockSpec((BM, E), lambda i: (i, 0)),
            pl.BlockSpec((1, 1, 1), lambda i: (i, 0, 0)),
        ],
        out_shape=[
            jax.ShapeDtypeStruct((M, E), jnp.float32),
            jax.ShapeDtypeStruct((M, E), jnp.float32),
            jax.ShapeDtypeStruct((N, 1, 1), jnp.float32),
        ],
    )(x, W, b.reshape(1, E))

    z_loss = jnp.sum(zparts) / M
    return probs.reshape(G, T, E), logits.reshape(G, T, E), z_loss


# speedup vs baseline: 1.0731x; 1.0731x over previous
import jax
import jax.numpy as jnp
from jax.experimental import pallas as pl


def _router_kernel(x_ref, w_ref, b_ref, probs_ref, logits_ref, z_ref):
    g = pl.program_id(0)
    t = pl.program_id(1)
    ng = pl.num_programs(0)
    nt = pl.num_programs(1)

    xb = x_ref[0].astype(jnp.bfloat16)
    wb = w_ref[...].astype(jnp.bfloat16)
    logits = jax.lax.dot_general(
        xb, wb,
        dimension_numbers=(((1,), (0,)), ((), ())),
        preferred_element_type=jnp.float32,
    )
    logits = logits + b_ref[...]
    logits_ref[0] = logits
    m = jnp.max(logits, axis=-1, keepdims=True)
    e = jnp.exp(logits - m)
    s = jnp.sum(e, axis=-1, keepdims=True)
    probs_ref[0] = e / s
    log_z = jnp.log(s) + m
    part = jnp.sum(log_z * log_z).reshape(1, 1)

    @pl.when((g == 0) & (t == 0))
    def _init():
        z_ref[...] = jnp.zeros((1, 1), jnp.float32)

    z_ref[...] += part

    @pl.when((g == ng - 1) & (t == nt - 1))
    def _norm():
        z_ref[...] = z_ref[...] * (1.0 / (ng * nt * x_ref.shape[1]))


def kernel(token_inputs, W, b, num_experts, expert_capacity):
    G, T, H = token_inputs.shape
    E = W.shape[1]
    BM = 1024

    probs, logits, z = pl.pallas_call(
        _router_kernel,
        grid=(G, T // BM),
        in_specs=[
            pl.BlockSpec((1, BM, H), lambda g, t: (g, t, 0)),
            pl.BlockSpec((H, E), lambda g, t: (0, 0)),
            pl.BlockSpec((1, E), lambda g, t: (0, 0)),
        ],
        out_specs=[
            pl.BlockSpec((1, BM, E), lambda g, t: (g, t, 0)),
            pl.BlockSpec((1, BM, E), lambda g, t: (g, t, 0)),
            pl.BlockSpec((1, 1), lambda g, t: (0, 0)),
        ],
        out_shape=[
            jax.ShapeDtypeStruct((G, T, E), jnp.float32),
            jax.ShapeDtypeStruct((G, T, E), jnp.float32),
            jax.ShapeDtypeStruct((1, 1), jnp.float32),
        ],
    )(token_inputs, W, b.reshape(1, E))

    return probs, logits, z[0, 0]


# speedup vs baseline: 1.2908x; 1.2028x over previous
import jax
import jax.numpy as jnp
from jax.experimental import pallas as pl


def _router_kernel(x_ref, w_ref, b_ref, probs_ref, logits_ref, z_ref):
    g = pl.program_id(0)
    t = pl.program_id(1)
    ng = pl.num_programs(0)
    nt = pl.num_programs(1)

    xb = x_ref[0].astype(jnp.bfloat16)
    wb = w_ref[...].astype(jnp.bfloat16)
    logits = jax.lax.dot_general(
        xb, wb,
        dimension_numbers=(((1,), (0,)), ((), ())),
        preferred_element_type=jnp.float32,
    )
    logits = logits + b_ref[...]
    logits_ref[0] = logits.T
    m = jnp.max(logits, axis=-1, keepdims=True)
    e = jnp.exp(logits - m)
    s = jnp.sum(e, axis=-1, keepdims=True)
    probs_ref[0] = (e / s).T
    log_z = jnp.log(s) + m
    part = jnp.sum(log_z * log_z).reshape(1, 1)

    @pl.when((g == 0) & (t == 0))
    def _init():
        z_ref[...] = jnp.zeros((1, 1), jnp.float32)

    z_ref[...] += part

    @pl.when((g == ng - 1) & (t == nt - 1))
    def _norm():
        z_ref[...] = z_ref[...] * (1.0 / (ng * nt * x_ref.shape[1]))


def kernel(token_inputs, W, b, num_experts, expert_capacity):
    G, T, H = token_inputs.shape
    E = W.shape[1]
    BM = 1024

    probs_t, logits_t, z = pl.pallas_call(
        _router_kernel,
        grid=(G, T // BM),
        in_specs=[
            pl.BlockSpec((1, BM, H), lambda g, t: (g, t, 0)),
            pl.BlockSpec((H, E), lambda g, t: (0, 0)),
            pl.BlockSpec((1, E), lambda g, t: (0, 0)),
        ],
        out_specs=[
            pl.BlockSpec((1, E, BM), lambda g, t: (g, 0, t)),
            pl.BlockSpec((1, E, BM), lambda g, t: (g, 0, t)),
            pl.BlockSpec((1, 1), lambda g, t: (0, 0)),
        ],
        out_shape=[
            jax.ShapeDtypeStruct((G, E, T), jnp.float32),
            jax.ShapeDtypeStruct((G, E, T), jnp.float32),
            jax.ShapeDtypeStruct((1, 1), jnp.float32),
        ],
    )(token_inputs, W, b.reshape(1, E))

    probs = jnp.transpose(probs_t, (0, 2, 1))
    logits = jnp.transpose(logits_t, (0, 2, 1))
    return probs, logits, z[0, 0]
